# baseline (speedup 1.0000x reference)
import functools

import jax
import jax.numpy as jnp
from jax import lax
from jax.experimental import pallas as pl
from jax.experimental.pallas import tpu as pltpu
from jax.experimental.pallas import tpu_sc as plsc

_EMB_DIM = 128
_B_POS = 16384
_B_NEG = 81920
_B_TOT = _B_POS + _B_NEG
_LANES = 16
_K = _EMB_DIM // _LANES
_GRP = _EMB_DIM // _LANES
_OUT_ROWS = _B_TOT * _LANES // _EMB_DIM

_NW = 32
_CHUNK = 128
_RPC = _CHUNK // _GRP
_POS_PER_W = _B_POS // _NW
_NEG_PER_W = _B_NEG // _NW
_POS_CHUNKS = _POS_PER_W // _CHUNK
_NEG_CHUNKS = _NEG_PER_W // _CHUNK
_N_CHUNKS = _POS_CHUNKS + _NEG_CHUNKS
_POS_ROWS = _POS_PER_W // _GRP
_NEG_ROWS = _NEG_PER_W // _GRP


@functools.cache
def _make_sc_scores():
    mesh = plsc.VectorSubcoreMesh(core_axis_name="c", subcore_axis_name="s")

    @functools.partial(
        pl.kernel,
        mesh=mesh,
        out_type=jax.ShapeDtypeStruct((_OUT_ROWS, _EMB_DIM), jnp.float32),
        scratch_types=[
            pltpu.VMEM((_N_CHUNKS * _CHUNK,), jnp.int32),
            pltpu.VMEM((_N_CHUNKS * _CHUNK,), jnp.int32),
            pltpu.VMEM((_CHUNK, _EMB_DIM), jnp.float32),
            pltpu.VMEM((_CHUNK, _EMB_DIM), jnp.float32),
            pltpu.VMEM((_CHUNK, _EMB_DIM), jnp.float32),
            pltpu.VMEM((_CHUNK, _EMB_DIM), jnp.float32),
            pltpu.VMEM((_CHUNK, _EMB_DIM), jnp.float32),
            pltpu.VMEM((_CHUNK, _EMB_DIM), jnp.float32),
            pltpu.VMEM((_RPC, _EMB_DIM), jnp.float32),
            pltpu.VMEM((_RPC, _EMB_DIM), jnp.float32),
            pltpu.VMEM((_RPC, _EMB_DIM), jnp.float32),
            pltpu.SemaphoreType.DMA,
            pltpu.SemaphoreType.DMA,
            pltpu.SemaphoreType.DMA,
            pltpu.SemaphoreType.DMA,
            pltpu.SemaphoreType.DMA,
            pltpu.SemaphoreType.DMA,
        ],
    )
    def sc_scores(pos_w1d, pos_v1d, neg_w1d, neg_v1d, w_emb, v_emb, out,
                  idx_a, idx_b, rows_a0, rows_b0, rows_a1, rows_b1,
                  rows_a2, rows_b2, part0, part1, part2,
                  sem0, sem1, sem2, semo0, semo1, semo2):
        wid = lax.axis_index("s") * 2 + lax.axis_index("c")
        rbufs = ((rows_a0, rows_b0, part0, sem0, semo0),
                 (rows_a1, rows_b1, part1, sem1, semo1),
                 (rows_a2, rows_b2, part2, sem2, semo2))

        pltpu.sync_copy(pos_w1d.at[pl.ds(wid * _POS_PER_W, _POS_PER_W)],
                        idx_a.at[pl.ds(0, _POS_PER_W)])
        pltpu.sync_copy(pos_v1d.at[pl.ds(wid * _POS_PER_W, _POS_PER_W)],
                        idx_b.at[pl.ds(0, _POS_PER_W)])
        pltpu.sync_copy(neg_w1d.at[pl.ds(wid * _NEG_PER_W, _NEG_PER_W)],
                        idx_a.at[pl.ds(_POS_PER_W, _NEG_PER_W)])
        pltpu.sync_copy(neg_v1d.at[pl.ds(wid * _NEG_PER_W, _NEG_PER_W)],
                        idx_b.at[pl.ds(_POS_PER_W, _NEG_PER_W)])

        def fetch(c, buf):
            rows_a, rows_b, _, sem, _ = buf
            ia = idx_a.at[pl.ds(c * _CHUNK, _CHUNK)]
            ib = idx_b.at[pl.ds(c * _CHUNK, _CHUNK)]

            @pl.when(c < _POS_CHUNKS)
            def _():
                pltpu.async_copy(w_emb.at[ia], rows_a, sem)

            @pl.when(c >= _POS_CHUNKS)
            def _():
                pltpu.async_copy(v_emb.at[ia], rows_a, sem)

            pltpu.async_copy(v_emb.at[ib], rows_b, sem)

        def consume(c, buf):
            rows_a, rows_b, part, sem, semo = buf
            ia = idx_a.at[pl.ds(c * _CHUNK, _CHUNK)]
            ib = idx_b.at[pl.ds(c * _CHUNK, _CHUNK)]
            pltpu.make_async_copy(v_emb.at[ia], rows_a, sem).wait()
            pltpu.make_async_copy(v_emb.at[ib], rows_b, sem).wait()

            @pl.when(c >= 3)
            def _():
                pltpu.make_async_copy(
                    part, out.at[pl.ds(0, _RPC), :], semo).wait()

            def row_body(rr, carry):
                for g in range(_GRP):
                    r = rr * _GRP + g
                    acc = (rows_a[r, pl.ds(0, _LANES)]
                           * rows_b[r, pl.ds(0, _LANES)])
                    for k in range(1, _K):
                        acc = acc + (rows_a[r, pl.ds(k * _LANES, _LANES)]
                                     * rows_b[r, pl.ds(k * _LANES, _LANES)])
                    part[rr, pl.ds(g * _LANES, _LANES)] = acc
                return carry

            lax.fori_loop(0, _RPC, row_body, 0)

            out_row = jnp.where(
                c < _POS_CHUNKS,
                wid * _POS_ROWS + c * _RPC,
                _B_POS // _GRP + wid * _NEG_ROWS + (c - _POS_CHUNKS) * _RPC)
            pltpu.async_copy(part, out.at[pl.ds(out_row, _RPC), :], semo)

        fetch(0, rbufs[0])
        fetch(1, rbufs[1])

        def body(i, carry):
            for b in range(3):
                cc = 3 * i + b
                nxt = cc + 2

                @pl.when(nxt < _N_CHUNKS)
                def _():
                    fetch(nxt, rbufs[(b + 2) % 3])

                consume(cc, rbufs[b])
            return carry

        lax.fori_loop(0, _N_CHUNKS // 3, body, 0)

        for b in range(3):
            _, _, part, _, semo = rbufs[b]
            pltpu.make_async_copy(part, out.at[pl.ds(0, _RPC), :], semo).wait()

    return sc_scores


def _tc_loss_body(x_ref, o_ref):
    x = x_ref[:]
    col = lax.broadcasted_iota(jnp.int32, (_EMB_DIM, _GRP), 0)
    grp = lax.broadcasted_iota(jnp.int32, (_EMB_DIM, _GRP), 1)
    sel = jnp.where(col // _LANES == grp, 1.0, 0.0).astype(jnp.float32)
    s = jnp.dot(x, sel, preferred_element_type=jnp.float32)
    s = jnp.clip(s, -10.0, 10.0)
    row = lax.broadcasted_iota(jnp.int32, s.shape, 0)
    sign = jnp.where(row < _B_POS // _GRP, 1.0, -1.0)
    ls = jax.nn.log_sigmoid(s * sign)
    o_ref[0, 0] = -jnp.sum(ls)


_tc_loss = pl.pallas_call(
    _tc_loss_body,
    out_shape=jax.ShapeDtypeStruct((1, 1), jnp.float32),
    out_specs=pl.BlockSpec(memory_space=pltpu.SMEM),
)


def kernel(pos_w, pos_v, neg_w, neg_v, w_embedding, v_embedding):
    scores16 = _make_sc_scores()(
        pos_w.astype(jnp.int32), pos_v.astype(jnp.int32),
        neg_w.astype(jnp.int32), neg_v.astype(jnp.int32),
        w_embedding, v_embedding)
    return _tc_loss(scores16)[0, 0]


# speedup vs baseline: 1.0216x; 1.0216x over previous
import functools

import jax
import jax.numpy as jnp
from jax import lax
from jax.experimental import pallas as pl
from jax.experimental.pallas import tpu as pltpu
from jax.experimental.pallas import tpu_sc as plsc

_EMB_DIM = 128
_B_POS = 16384
_B_NEG = 81920
_B_TOT = _B_POS + _B_NEG
_LANES = 16
_K = _EMB_DIM // _LANES
_GRP = _EMB_DIM // _LANES
_OUT_ROWS = _B_TOT * _LANES // _EMB_DIM

_NW = 32
_CHUNK = 128
_RPC = _CHUNK // _GRP
_POS_PER_W = _B_POS // _NW
_NEG_PER_W = _B_NEG // _NW
_POS_CHUNKS = _POS_PER_W // _CHUNK
_NEG_CHUNKS = _NEG_PER_W // _CHUNK
_N_CHUNKS = _POS_CHUNKS + _NEG_CHUNKS
_POS_ROWS = _POS_PER_W // _GRP
_NEG_ROWS = _NEG_PER_W // _GRP


@functools.cache
def _make_sc_scores():
    mesh = plsc.VectorSubcoreMesh(core_axis_name="c", subcore_axis_name="s")

    @functools.partial(
        pl.kernel,
        mesh=mesh,
        out_type=jax.ShapeDtypeStruct((_OUT_ROWS, _EMB_DIM), jnp.float32),
        scratch_types=[
            pltpu.VMEM((_N_CHUNKS * _CHUNK,), jnp.int32),
            pltpu.VMEM((_N_CHUNKS * _CHUNK,), jnp.int32),
            pltpu.VMEM((_CHUNK, _EMB_DIM), jnp.float32),
            pltpu.VMEM((_CHUNK, _EMB_DIM), jnp.float32),
            pltpu.VMEM((_CHUNK, _EMB_DIM), jnp.float32),
            pltpu.VMEM((_CHUNK, _EMB_DIM), jnp.float32),
            pltpu.VMEM((_CHUNK, _EMB_DIM), jnp.float32),
            pltpu.VMEM((_CHUNK, _EMB_DIM), jnp.float32),
            pltpu.VMEM((_RPC, _EMB_DIM), jnp.float32),
            pltpu.VMEM((_RPC, _EMB_DIM), jnp.float32),
            pltpu.VMEM((_RPC, _EMB_DIM), jnp.float32),
            pltpu.SemaphoreType.DMA,
            pltpu.SemaphoreType.DMA,
            pltpu.SemaphoreType.DMA,
            pltpu.SemaphoreType.DMA,
            pltpu.SemaphoreType.DMA,
            pltpu.SemaphoreType.DMA,
        ],
    )
    def sc_scores(pos_w1d, pos_v1d, neg_w1d, neg_v1d, w_emb, v_emb, out,
                  idx_a, idx_b, rows_a0, rows_b0, rows_a1, rows_b1,
                  rows_a2, rows_b2, part0, part1, part2,
                  sem0, sem1, sem2, semo0, semo1, semo2):
        wid = lax.axis_index("s") * 2 + lax.axis_index("c")
        rbufs = ((rows_a0, rows_b0, part0, sem0, semo0),
                 (rows_a1, rows_b1, part1, sem1, semo1),
                 (rows_a2, rows_b2, part2, sem2, semo2))

        stages = (
            (pos_w1d.at[pl.ds(wid * _POS_PER_W, _POS_PER_W)],
             idx_a.at[pl.ds(0, _POS_PER_W)]),
            (pos_v1d.at[pl.ds(wid * _POS_PER_W, _POS_PER_W)],
             idx_b.at[pl.ds(0, _POS_PER_W)]),
            (neg_w1d.at[pl.ds(wid * _NEG_PER_W, _NEG_PER_W)],
             idx_a.at[pl.ds(_POS_PER_W, _NEG_PER_W)]),
            (neg_v1d.at[pl.ds(wid * _NEG_PER_W, _NEG_PER_W)],
             idx_b.at[pl.ds(_POS_PER_W, _NEG_PER_W)]),
        )
        cps = [pltpu.async_copy(src, dst, semo0) for src, dst in stages]
        for cp in cps:
            cp.wait()

        def fetch(c, buf):
            rows_a, rows_b, _, sem, _ = buf
            ia = idx_a.at[pl.ds(c * _CHUNK, _CHUNK)]
            ib = idx_b.at[pl.ds(c * _CHUNK, _CHUNK)]

            @pl.when(c < _POS_CHUNKS)
            def _():
                pltpu.async_copy(w_emb.at[ia], rows_a, sem)

            @pl.when(c >= _POS_CHUNKS)
            def _():
                pltpu.async_copy(v_emb.at[ia], rows_a, sem)

            pltpu.async_copy(v_emb.at[ib], rows_b, sem)

        def consume(c, buf):
            rows_a, rows_b, part, sem, semo = buf
            ia = idx_a.at[pl.ds(c * _CHUNK, _CHUNK)]
            ib = idx_b.at[pl.ds(c * _CHUNK, _CHUNK)]
            pltpu.make_async_copy(v_emb.at[ia], rows_a, sem).wait()
            pltpu.make_async_copy(v_emb.at[ib], rows_b, sem).wait()

            @pl.when(c >= 3)
            def _():
                pltpu.make_async_copy(
                    part, out.at[pl.ds(0, _RPC), :], semo).wait()

            def row_body(rr, carry):
                for g in range(_GRP):
                    r = rr * _GRP + g
                    acc = (rows_a[r, pl.ds(0, _LANES)]
                           * rows_b[r, pl.ds(0, _LANES)])
                    for k in range(1, _K):
                        acc = acc + (rows_a[r, pl.ds(k * _LANES, _LANES)]
                                     * rows_b[r, pl.ds(k * _LANES, _LANES)])
                    part[rr, pl.ds(g * _LANES, _LANES)] = acc
                return carry

            lax.fori_loop(0, _RPC, row_body, 0)

            out_row = jnp.where(
                c < _POS_CHUNKS,
                wid * _POS_ROWS + c * _RPC,
                _B_POS // _GRP + wid * _NEG_ROWS + (c - _POS_CHUNKS) * _RPC)
            pltpu.async_copy(part, out.at[pl.ds(out_row, _RPC), :], semo)

        fetch(0, rbufs[0])
        fetch(1, rbufs[1])

        def body(i, carry):
            for b in range(3):
                cc = 3 * i + b
                nxt = cc + 2

                @pl.when(nxt < _N_CHUNKS)
                def _():
                    fetch(nxt, rbufs[(b + 2) % 3])

                consume(cc, rbufs[b])
            return carry

        lax.fori_loop(0, _N_CHUNKS // 3, body, 0)

        for b in range(3):
            _, _, part, _, semo = rbufs[b]
            pltpu.make_async_copy(part, out.at[pl.ds(0, _RPC), :], semo).wait()

    return sc_scores


def _tc_loss_body(x_ref, o_ref):
    x = x_ref[:]
    col = lax.broadcasted_iota(jnp.int32, (_EMB_DIM, _GRP), 0)
    grp = lax.broadcasted_iota(jnp.int32, (_EMB_DIM, _GRP), 1)
    sel = jnp.where(col // _LANES == grp, 1.0, 0.0).astype(jnp.float32)
    s = jnp.dot(x, sel, preferred_element_type=jnp.float32)
    s = jnp.clip(s, -10.0, 10.0)
    row = lax.broadcasted_iota(jnp.int32, s.shape, 0)
    sign = jnp.where(row < _B_POS // _GRP, 1.0, -1.0)
    ls = jax.nn.log_sigmoid(s * sign)
    o_ref[0, 0] = -jnp.sum(ls)


_tc_loss = pl.pallas_call(
    _tc_loss_body,
    out_shape=jax.ShapeDtypeStruct((1, 1), jnp.float32),
    out_specs=pl.BlockSpec(memory_space=pltpu.SMEM),
)


def kernel(pos_w, pos_v, neg_w, neg_v, w_embedding, v_embedding):
    scores16 = _make_sc_scores()(
        pos_w.astype(jnp.int32), pos_v.astype(jnp.int32),
        neg_w.astype(jnp.int32), neg_v.astype(jnp.int32),
        w_embedding, v_embedding)
    return _tc_loss(scores16)[0, 0]
